# baseline (speedup 1.0000x reference)
import functools

import jax
import jax.numpy as jnp
from jax import lax
from jax.experimental import pallas as pl
from jax.experimental.pallas import tpu as pltpu
from jax.experimental.pallas import tpu_sc as plsc

B = 16384
D = 64
NC = 2
NS = 16
NW = NC * NS
N_PER_W = B // NW
CHUNK = 256
N_CHUNKS = N_PER_W // CHUNK
LANES = 16


def _sc_body(h_hbm, r_hbm, t_hbm, rel_hbm, table_hbm, out_hbm,
             idx_v, rows_v, h_v, r_v, t_v, out_v, sem, gsem):
    wid = lax.axis_index("s") * NC + lax.axis_index("c")
    base = wid * N_PER_W
    for c in range(N_CHUNKS):
        col0 = base + c * CHUNK
        pltpu.sync_copy(rel_hbm.at[pl.ds(col0, CHUNK)], idx_v)
        dense = [
            pltpu.async_copy(h_hbm.at[:, pl.ds(col0, CHUNK)], h_v, sem),
            pltpu.async_copy(r_hbm.at[:, pl.ds(col0, CHUNK)], r_v, sem),
            pltpu.async_copy(t_hbm.at[:, pl.ds(col0, CHUNK)], t_v, sem),
        ]
        gathers = [
            pltpu.async_copy(table_hbm.at[d].at[idx_v], rows_v.at[d], gsem)
            for d in range(D)
        ]
        for cp in dense:
            cp.wait()
        for cp in gathers:
            cp.wait()

        def group_body(g, carry, c=c):
            sl = pl.ds(g * LANES, LANES)
            acc = jnp.zeros((LANES,), jnp.float32)
            for d in range(D):
                acc = acc + h_v[d, sl] * t_v[d, sl] * (r_v[d, sl]
                                                       + rows_v[d, sl])
            out_v[pl.ds(c * CHUNK + g * LANES, LANES)] = acc * 0.5
            return carry

        lax.fori_loop(0, CHUNK // LANES, group_body, 0)
    pltpu.sync_copy(out_v, out_hbm.at[pl.ds(base, N_PER_W)])


@functools.partial(
    pl.kernel,
    out_type=jax.ShapeDtypeStruct((B,), jnp.float32),
    mesh=plsc.VectorSubcoreMesh(core_axis_name="c", subcore_axis_name="s"),
    compiler_params=pltpu.CompilerParams(
        needs_layout_passes=False, use_tc_tiling_on_sc=False),
    scratch_types=[
        pltpu.VMEM((CHUNK,), jnp.int32),
        pltpu.VMEM((D, CHUNK), jnp.float32),
        pltpu.VMEM((D, CHUNK), jnp.float32),
        pltpu.VMEM((D, CHUNK), jnp.float32),
        pltpu.VMEM((D, CHUNK), jnp.float32),
        pltpu.VMEM((N_PER_W,), jnp.float32),
        pltpu.SemaphoreType.DMA,
        pltpu.SemaphoreType.DMA,
    ],
)
def _simple_cal_sc(h_hbm, r_hbm, t_hbm, rel_hbm, table_hbm, out_hbm,
                   idx_v, rows_v, h_v, r_v, t_v, out_v, sem, gsem):
    _sc_body(h_hbm, r_hbm, t_hbm, rel_hbm, table_hbm, out_hbm,
             idx_v, rows_v, h_v, r_v, t_v, out_v, sem, gsem)


def kernel(x0, x1, x2, rel, rel_inv_table):
    h = x0.reshape(B, D).T
    r = x1.reshape(B, D).T
    t = x2.reshape(B, D).T
    table = rel_inv_table.T
    out = _simple_cal_sc(h, r, t, rel, table)
    return out[:, None]


# speedup vs baseline: 7.9060x; 7.9060x over previous
import functools

import jax
import jax.numpy as jnp
from jax import lax
from jax.experimental import pallas as pl
from jax.experimental.pallas import tpu as pltpu
from jax.experimental.pallas import tpu_sc as plsc

B = 16384
D = 64
NC = 2
NS = 16
NW = NC * NS
N_PER_W = B // NW
CHUNK = 256
N_CHUNKS = N_PER_W // CHUNK
LANES = 16


def _sc_body(h_hbm, r_hbm, t_hbm, rel_hbm, table_hbm, out_hbm,
             idx_v, rows_v, h_v, r_v, t_v, out_v, sem, gsem):
    wid = lax.axis_index("s") * NC + lax.axis_index("c")
    base = wid * N_PER_W
    for c in range(N_CHUNKS):
        col0 = base + c * CHUNK
        pltpu.sync_copy(rel_hbm.at[pl.ds(col0, CHUNK)], idx_v)
        gather = pltpu.async_copy(table_hbm.at[idx_v], rows_v, gsem)
        dense = [
            pltpu.async_copy(h_hbm.at[:, pl.ds(col0, CHUNK)], h_v, sem),
            pltpu.async_copy(r_hbm.at[:, pl.ds(col0, CHUNK)], r_v, sem),
            pltpu.async_copy(t_hbm.at[:, pl.ds(col0, CHUNK)], t_v, sem),
        ]
        for cp in dense:
            cp.wait()
        gather.wait()

        def group_body(g, carry, c=c):
            sl = pl.ds(g * LANES, LANES)
            bvec = lax.iota(jnp.int32, LANES) + g * LANES
            acc = jnp.zeros((LANES,), jnp.float32)
            for d in range(D):
                dvec = jnp.full((LANES,), d, jnp.int32)
                gv = plsc.load_gather(rows_v, [bvec, dvec])
                acc = acc + h_v[d, sl] * t_v[d, sl] * (r_v[d, sl] + gv)
            out_v[pl.ds(c * CHUNK + g * LANES, LANES)] = acc * 0.5
            return carry

        lax.fori_loop(0, CHUNK // LANES, group_body, 0)
    pltpu.sync_copy(out_v, out_hbm.at[pl.ds(base, N_PER_W)])


@functools.partial(
    pl.kernel,
    out_type=jax.ShapeDtypeStruct((B,), jnp.float32),
    mesh=plsc.VectorSubcoreMesh(core_axis_name="c", subcore_axis_name="s"),
    compiler_params=pltpu.CompilerParams(
        needs_layout_passes=False, use_tc_tiling_on_sc=False),
    scratch_types=[
        pltpu.VMEM((CHUNK,), jnp.int32),
        pltpu.VMEM((CHUNK, D), jnp.float32),
        pltpu.VMEM((D, CHUNK), jnp.float32),
        pltpu.VMEM((D, CHUNK), jnp.float32),
        pltpu.VMEM((D, CHUNK), jnp.float32),
        pltpu.VMEM((N_PER_W,), jnp.float32),
        pltpu.SemaphoreType.DMA,
        pltpu.SemaphoreType.DMA,
    ],
)
def _simple_cal_sc(h_hbm, r_hbm, t_hbm, rel_hbm, table_hbm, out_hbm,
                   idx_v, rows_v, h_v, r_v, t_v, out_v, sem, gsem):
    _sc_body(h_hbm, r_hbm, t_hbm, rel_hbm, table_hbm, out_hbm,
             idx_v, rows_v, h_v, r_v, t_v, out_v, sem, gsem)


def kernel(x0, x1, x2, rel, rel_inv_table):
    h = x0.reshape(B, D).T
    r = x1.reshape(B, D).T
    t = x2.reshape(B, D).T
    out = _simple_cal_sc(h, r, t, rel, rel_inv_table)
    return out[:, None]
